# baseline (speedup 1.0000x reference)
import jax
import jax.numpy as jnp
from jax import lax
from jax.experimental import pallas as pl
from jax.experimental.pallas import tpu as pltpu
import jax.experimental.pallas.tpu_sc as plsc

N = 98304
D = 256
S = 32768
NC = 2
NS = 16
NW = NC * NS
BINS_PER_W = N // NW
CAP = 4096
PAD = 2048
L = 16

ROWS_PER_W = N // NW
WIN = 128
N_WIN = ROWS_PER_W // WIN

_I17 = (1 << 17) - 1


def _rank_body(keys_hbm, rank_hbm, keys_v, coll_v, ibuf_v, pbuf_v,
               hist_v, start_v, sem):
    wid = lax.axis_index("s") * NC + lax.axis_index("c")
    lo = wid * BINS_PER_W
    hi = lo + BINS_PER_W
    iota = jnp.arange(L, dtype=jnp.int32)
    ones = jnp.ones((L,), jnp.int32)
    zeros = jnp.zeros((L,), jnp.int32)

    pltpu.sync_copy(keys_hbm, keys_v)

    def zbody(j, _):
        hist_v[pl.ds(j * L, L)] = zeros
        return 0
    lax.fori_loop(0, BINS_PER_W // L, zbody, 0)

    def scan(j, carry):
        off, below = carry
        v = keys_v[pl.ds(j * L, L)]
        in_mask = (v >= lo) & (v < hi)
        below = below + (v < lo).astype(jnp.int32)
        packed = ((v - lo) << 17) | (j * L + iota)
        plsc.store_compressed(coll_v.at[pl.ds(off, L)], packed, mask=in_mask)
        off = off + jnp.sum(in_mask.astype(jnp.int32))
        return off, below

    m_t, below_vec = lax.fori_loop(0, N // L, scan, (jnp.int32(0), zeros))
    base = jnp.sum(below_vec)
    nv = (m_t + L - 1) // L

    def hbody(j, _):
        p = coll_v[pl.ds(j * L, L)]
        valid = (j * L + iota) < m_t
        b = jnp.where(valid, lax.shift_right_logical(p, 17), 0)
        plsc.addupdate_scatter(hist_v, [b], ones, mask=valid)
        return 0
    lax.fori_loop(0, nv, hbody, 0)

    def cbody(j, carry):
        h = hist_v[pl.ds(j * L, L)]
        cs = plsc.cumsum(h)
        start_v[pl.ds(j * L, L)] = cs - h + carry
        return carry + jnp.sum(h)
    lax.fori_loop(0, BINS_PER_W // L, cbody, jnp.int32(0))

    def fbody(j, _):
        ibuf_v[pl.ds(j * L, L)] = N + ((wid * 64 + j * L + iota) & (PAD - 1))
        return 0
    lax.fori_loop(0, CAP // L, fbody, 0)

    def pbody(j, _):
        p = coll_v[pl.ds(j * L, L)]
        valid = (j * L + iota) < m_t
        b = jnp.where(valid, lax.shift_right_logical(p, 17), 0)
        i = p & _I17
        occ, _ = plsc.scan_count(b, mask=valid)
        pos = base + plsc.load_gather(start_v, [b]) + occ - 1
        plsc.addupdate_scatter(start_v, [b], ones, mask=valid)
        sl = j * L + iota
        plsc.store_scatter(ibuf_v, [sl], i, mask=valid)
        plsc.store_scatter(pbuf_v, [sl], pos, mask=valid)
        return 0
    lax.fori_loop(0, nv, pbody, 0)

    pltpu.async_copy(pbuf_v, rank_hbm.at[ibuf_v], sem).wait()


def _rows_body(asc_hbm, cru_hbm, des_hbm, rank_hbm, out_hbm,
               idx_v, rows_v, sem):
    wid = lax.axis_index("s") * NC + lax.axis_index("c")
    row0 = wid * ROWS_PER_W

    def win(w, _):
        start = row0 + w * WIN
        pltpu.sync_copy(rank_hbm.at[pl.ds(start, WIN)], idx_v)
        src = start // S
        local = start - src * S

        @pl.when(src == 0)
        def _():
            pltpu.sync_copy(asc_hbm.at[pl.ds(local, WIN)], rows_v)

        @pl.when(src == 1)
        def _():
            pltpu.sync_copy(cru_hbm.at[pl.ds(local, WIN)], rows_v)

        @pl.when(src == 2)
        def _():
            pltpu.sync_copy(des_hbm.at[pl.ds(local, WIN)], rows_v)

        pltpu.async_copy(rows_v, out_hbm.at[idx_v], sem).wait()
        return 0

    lax.fori_loop(0, N_WIN, win, 0)


def kernel(asc_dec, cru_dec, des_dec, concat_index):
    mesh = plsc.VectorSubcoreMesh(core_axis_name="c", subcore_axis_name="s")

    rank_k = pl.kernel(
        _rank_body,
        mesh=mesh,
        out_type=jax.ShapeDtypeStruct((N + PAD,), jnp.int32),
        scratch_types=[
            pltpu.VMEM((N,), jnp.int32),
            pltpu.VMEM((CAP,), jnp.int32),
            pltpu.VMEM((CAP,), jnp.int32),
            pltpu.VMEM((CAP,), jnp.int32),
            pltpu.VMEM((BINS_PER_W,), jnp.int32),
            pltpu.VMEM((BINS_PER_W,), jnp.int32),
            pltpu.SemaphoreType.DMA,
        ],
        compiler_params=pltpu.CompilerParams(needs_layout_passes=False),
    )
    rank = rank_k(concat_index)

    rows_k = pl.kernel(
        _rows_body,
        mesh=mesh,
        out_type=jax.ShapeDtypeStruct((N, D), jnp.float32),
        scratch_types=[
            pltpu.VMEM((WIN,), jnp.int32),
            pltpu.VMEM((WIN, D), jnp.float32),
            pltpu.SemaphoreType.DMA,
        ],
    )
    return rows_k(asc_dec, cru_dec, des_dec, rank)


# speedup vs baseline: 1.0863x; 1.0863x over previous
import jax
import jax.numpy as jnp
from jax import lax
from jax.experimental import pallas as pl
from jax.experimental.pallas import tpu as pltpu
import jax.experimental.pallas.tpu_sc as plsc

N = 98304
D = 256
S = 32768
NC = 2
NS = 16
NW = NC * NS
BINS_PER_W = N // NW
CAP = 4096
PAD = 2048
L = 16

ROWS_PER_W = N // NW
WIN = 128
N_WIN = ROWS_PER_W // WIN

_I17 = (1 << 17) - 1


def _rank_body(keys_hbm, rank_hbm, keys_v, coll_v, ibuf_v, pbuf_v,
               hist_v, start_v, sem):
    wid = lax.axis_index("s") * NC + lax.axis_index("c")
    lo = wid * BINS_PER_W
    hi = lo + BINS_PER_W
    iota = jnp.arange(L, dtype=jnp.int32)
    ones = jnp.ones((L,), jnp.int32)
    zeros = jnp.zeros((L,), jnp.int32)

    pltpu.sync_copy(keys_hbm, keys_v)

    @plsc.parallel_loop(0, BINS_PER_W // L, unroll=8)
    def _zero(j):
        hist_v[pl.ds(j * L, L)] = zeros

    @plsc.parallel_loop(0, N // L, unroll=8, carry=(jnp.int32(0), zeros))
    def scan(j, carry):
        off, below = carry
        v = keys_v[pl.ds(j * L, L)]
        in_mask = (v >= lo) & (v < hi)
        below = below + (v < lo).astype(jnp.int32)
        packed = ((v - lo) << 17) | (j * L + iota)
        plsc.store_compressed(coll_v.at[pl.ds(off, L)], packed, mask=in_mask)
        off = off + jnp.sum(in_mask.astype(jnp.int32))
        return off, below

    m_t, below_vec = scan
    base = jnp.sum(below_vec)
    nv = (m_t + L - 1) // L

    def hbody(j, _):
        p = coll_v[pl.ds(j * L, L)]
        valid = (j * L + iota) < m_t
        b = jnp.where(valid, lax.shift_right_logical(p, 17), 0)
        plsc.addupdate_scatter(hist_v, [b], ones, mask=valid)
        return 0
    lax.fori_loop(0, nv, hbody, 0)

    @plsc.parallel_loop(0, BINS_PER_W // L, unroll=8, carry=jnp.int32(0))
    def _cum(j, carry):
        h = hist_v[pl.ds(j * L, L)]
        cs = plsc.cumsum(h)
        start_v[pl.ds(j * L, L)] = cs - h + carry
        return carry + jnp.sum(h)

    @plsc.parallel_loop(0, CAP // L, unroll=8)
    def _fill(j):
        ibuf_v[pl.ds(j * L, L)] = N + ((wid * 64 + j * L + iota) & (PAD - 1))

    def pbody(j, _):
        p = coll_v[pl.ds(j * L, L)]
        valid = (j * L + iota) < m_t
        b = jnp.where(valid, lax.shift_right_logical(p, 17), 0)
        i = p & _I17
        occ, _ = plsc.scan_count(b, mask=valid)
        pos = base + plsc.load_gather(start_v, [b]) + occ - 1
        plsc.addupdate_scatter(start_v, [b], ones, mask=valid)
        sl = j * L + iota
        plsc.store_scatter(ibuf_v, [sl], i, mask=valid)
        plsc.store_scatter(pbuf_v, [sl], pos, mask=valid)
        return 0
    lax.fori_loop(0, nv, pbody, 0)

    pltpu.async_copy(pbuf_v, rank_hbm.at[ibuf_v], sem).wait()


def _rows_body(asc_hbm, cru_hbm, des_hbm, rank_hbm, out_hbm,
               idx_v, rows_v, sem):
    wid = lax.axis_index("s") * NC + lax.axis_index("c")
    row0 = wid * ROWS_PER_W

    def win(w, _):
        start = row0 + w * WIN
        pltpu.sync_copy(rank_hbm.at[pl.ds(start, WIN)], idx_v)
        src = start // S
        local = start - src * S

        @pl.when(src == 0)
        def _():
            pltpu.sync_copy(asc_hbm.at[pl.ds(local, WIN)], rows_v)

        @pl.when(src == 1)
        def _():
            pltpu.sync_copy(cru_hbm.at[pl.ds(local, WIN)], rows_v)

        @pl.when(src == 2)
        def _():
            pltpu.sync_copy(des_hbm.at[pl.ds(local, WIN)], rows_v)

        pltpu.async_copy(rows_v, out_hbm.at[idx_v], sem).wait()
        return 0

    lax.fori_loop(0, N_WIN, win, 0)


def kernel(asc_dec, cru_dec, des_dec, concat_index):
    mesh = plsc.VectorSubcoreMesh(core_axis_name="c", subcore_axis_name="s")

    rank_k = pl.kernel(
        _rank_body,
        mesh=mesh,
        out_type=jax.ShapeDtypeStruct((N + PAD,), jnp.int32),
        scratch_types=[
            pltpu.VMEM((N,), jnp.int32),
            pltpu.VMEM((CAP,), jnp.int32),
            pltpu.VMEM((CAP,), jnp.int32),
            pltpu.VMEM((CAP,), jnp.int32),
            pltpu.VMEM((BINS_PER_W,), jnp.int32),
            pltpu.VMEM((BINS_PER_W,), jnp.int32),
            pltpu.SemaphoreType.DMA,
        ],
        compiler_params=pltpu.CompilerParams(needs_layout_passes=False),
    )
    rank = rank_k(concat_index)

    rows_k = pl.kernel(
        _rows_body,
        mesh=mesh,
        out_type=jax.ShapeDtypeStruct((N, D), jnp.float32),
        scratch_types=[
            pltpu.VMEM((WIN,), jnp.int32),
            pltpu.VMEM((WIN, D), jnp.float32),
            pltpu.SemaphoreType.DMA,
        ],
    )
    return rows_k(asc_dec, cru_dec, des_dec, rank)
